# Initial kernel scaffold; baseline (speedup 1.0000x reference)
import jax
import jax.numpy as jnp
from jax.experimental import pallas as pl


def kernel(x, edge_weight, W1, b1, W2, b2, W3, b3, Wmu, bmu, Wlv, blv, Wd1, bd1, Wd2, bd2, edge_index, beta):
    raise NotImplementedError("write your pallas kernel here")


# speedup vs baseline: 2.1106x; 2.1106x over previous
import functools

import jax
import jax.numpy as jnp
from jax import lax
from jax.experimental import pallas as pl
from jax.experimental.pallas import tpu as pltpu
from jax.experimental.pallas import tpu_sc as plsc

N = 10000
NPAD = 10240
E = 320000
NTILE = 32
KB = 128
NB = 79
EPAD = NTILE * NB * KB
MB = 1024
GRID_M = NPAD // MB


def _prep_kernel(degp_ref, x_ref, dis_ref, xt_ref):
    d = 1.0 + jnp.sum(degp_ref[...], axis=0)[:, None]
    dis = lax.rsqrt(d)
    dis_ref[...] = dis
    xt_ref[...] = x_ref[...] * dis


def _tc_prep(degp, x_pad):
    return pl.pallas_call(
        _prep_kernel,
        grid=(GRID_M,),
        in_specs=[
            pl.BlockSpec((NTILE, MB), lambda m: (0, m)),
            pl.BlockSpec((MB, 128), lambda m: (m, 0)),
        ],
        out_specs=[
            pl.BlockSpec((MB, 1), lambda m: (m, 0)),
            pl.BlockSpec((MB, 128), lambda m: (m, 0)),
        ],
        out_shape=[
            jax.ShapeDtypeStruct((NPAD, 1), jnp.float32),
            jax.ShapeDtypeStruct((NPAD, 128), jnp.float32),
        ],
    )(degp, x_pad)


def _post_kernel(nc, p_ref, xt_ref, dis_ref, w_ref, b_ref, h_ref):
    dis = dis_ref[...]
    cols = []
    for c in range(nc):
        cols.append(dis * (p_ref[0, c] + p_ref[1, c] + xt_ref[c]))
    y = jnp.concatenate(cols, axis=-1)
    if w_ref is not None:
        y = jnp.dot(y, w_ref[...], preferred_element_type=jnp.float32)
    y = y + b_ref[...]
    nrm = jnp.sqrt(jnp.sum(y * y, axis=-1, keepdims=True))
    y = y / jnp.clip(nrm, 1e-12, None)
    h_ref[...] = jnp.maximum(y, 0.0)


def _tc_post(p, xt, dis, b, w=None):
    nc = p.shape[1]
    fout = w.shape[1] if w is not None else nc * 128
    kern = functools.partial(_post_kernel, nc)
    in_specs = [
        pl.BlockSpec((2, nc, MB, 128), lambda m: (0, 0, m, 0)),
        pl.BlockSpec((nc, MB, 128), lambda m: (0, m, 0)),
        pl.BlockSpec((MB, 1), lambda m: (m, 0)),
    ]
    args = [p, xt, dis]
    if w is not None:
        in_specs.append(pl.BlockSpec(w.shape, lambda m: (0, 0)))
        args.append(w)
    else:
        kern = functools.partial(_post_kernel_nw, nc)
    in_specs.append(pl.BlockSpec((1, fout), lambda m: (0, 0)))
    args.append(b.reshape(1, fout))
    return pl.pallas_call(
        kern,
        grid=(GRID_M,),
        in_specs=in_specs,
        out_specs=pl.BlockSpec((MB, fout), lambda m: (m, 0)),
        out_shape=jax.ShapeDtypeStruct((NPAD, fout), jnp.float32),
    )(*args)


def _post_kernel_nw(nc, p_ref, xt_ref, dis_ref, b_ref, h_ref):
    _post_kernel(nc, p_ref, xt_ref, dis_ref, None, b_ref, h_ref)


def _mm_chunk_kernel(h_ref, w_ref, dis_ref, o_ref):
    g = jnp.dot(h_ref[...], w_ref[...], preferred_element_type=jnp.float32)
    o_ref[0] = dis_ref[...] * g


def _tc_mm_chunk(h, w, dis):
    kdim = h.shape[1]
    nc = w.shape[1] // 128
    return pl.pallas_call(
        _mm_chunk_kernel,
        grid=(GRID_M, nc),
        in_specs=[
            pl.BlockSpec((MB, kdim), lambda m, c: (m, 0)),
            pl.BlockSpec((kdim, 128), lambda m, c: (0, c)),
            pl.BlockSpec((MB, 1), lambda m, c: (m, 0)),
        ],
        out_specs=pl.BlockSpec((1, MB, 128), lambda m, c: (c, m, 0)),
        out_shape=jax.ShapeDtypeStruct((nc, NPAD, 128), jnp.float32),
    )(h, w, dis)


def _head_kernel(p_ref, xt_ref, dis_ref, b_ref, wmu_ref, bmu_ref, wlv_ref,
                 blv_ref, eps_ref, sb_ref, mu_ref, lv_ref, zmax_ref, zsum_ref):
    nc = 2
    dis = dis_ref[...]
    cols = []
    for c in range(nc):
        cols.append(dis * (p_ref[0, c] + p_ref[1, c] + xt_ref[c]))
    y = jnp.concatenate(cols, axis=-1) + b_ref[...]
    nrm = jnp.sqrt(jnp.sum(y * y, axis=-1, keepdims=True))
    h = jnp.maximum(y / jnp.clip(nrm, 1e-12, None), 0.0)
    mu = jnp.dot(h, wmu_ref[...], preferred_element_type=jnp.float32) + bmu_ref[...]
    lv = jnp.dot(h, wlv_ref[...], preferred_element_type=jnp.float32) + blv_ref[...]
    mu_ref[...] = mu
    lv_ref[...] = lv
    z = mu + eps_ref[...] * jnp.exp(sb_ref[0, 0] * lv)
    m = pl.program_id(0)
    rows = m * MB + lax.broadcasted_iota(jnp.int32, (MB, 1), 0)
    valid = rows < N
    zmax = jnp.max(jnp.where(valid, z, -jnp.inf), axis=0, keepdims=True)
    zsum = jnp.sum(jnp.where(valid, z, 0.0), axis=0, keepdims=True)
    zmax_b = jnp.broadcast_to(zmax, (8, zmax.shape[1]))
    zsum_b = jnp.broadcast_to(zsum, (8, zsum.shape[1]))

    @pl.when(m == 0)
    def _():
        zmax_ref[...] = zmax_b
        zsum_ref[...] = zsum_b

    @pl.when(m > 0)
    def _():
        zmax_ref[...] = jnp.maximum(zmax_ref[...], zmax_b)
        zsum_ref[...] = zsum_ref[...] + zsum_b


def _tc_head(p, xt, dis, b3, wmu, bmu, wlv, blv, eps_pad, sb):
    return pl.pallas_call(
        _head_kernel,
        grid=(GRID_M,),
        in_specs=[
            pl.BlockSpec((2, 2, MB, 128), lambda m: (0, 0, m, 0)),
            pl.BlockSpec((2, MB, 128), lambda m: (0, m, 0)),
            pl.BlockSpec((MB, 1), lambda m: (m, 0)),
            pl.BlockSpec((1, 256), lambda m: (0, 0)),
            pl.BlockSpec((256, 512), lambda m: (0, 0)),
            pl.BlockSpec((1, 512), lambda m: (0, 0)),
            pl.BlockSpec((256, 512), lambda m: (0, 0)),
            pl.BlockSpec((1, 512), lambda m: (0, 0)),
            pl.BlockSpec((MB, 512), lambda m: (m, 0)),
            pl.BlockSpec(memory_space=pltpu.SMEM),
        ],
        out_specs=[
            pl.BlockSpec((MB, 512), lambda m: (m, 0)),
            pl.BlockSpec((MB, 512), lambda m: (m, 0)),
            pl.BlockSpec((8, 512), lambda m: (0, 0)),
            pl.BlockSpec((8, 512), lambda m: (0, 0)),
        ],
        out_shape=[
            jax.ShapeDtypeStruct((NPAD, 512), jnp.float32),
            jax.ShapeDtypeStruct((NPAD, 512), jnp.float32),
            jax.ShapeDtypeStruct((8, 512), jnp.float32),
            jax.ShapeDtypeStruct((8, 512), jnp.float32),
        ],
    )(p, xt, dis, b3.reshape(1, 256), wmu, bmu.reshape(1, 512), wlv,
      blv.reshape(1, 512), eps_pad, sb)


def _dec_kernel(zmax_ref, zsum_ref, wd1_ref, bd1_ref, wd2_ref, bd2_ref, o_ref):
    rz = jnp.concatenate([zmax_ref[...], zsum_ref[...] / N], axis=-1)
    hd = jnp.dot(rz, wd1_ref[...], preferred_element_type=jnp.float32) + bd1_ref[...]
    hd = jnp.maximum(hd, 0.0)
    o = jnp.dot(hd, wd2_ref[...], preferred_element_type=jnp.float32) + bd2_ref[...]
    o_ref[...] = jax.nn.sigmoid(o)


def _tc_dec(zmax, zsum, wd1, bd1, wd2, bd2):
    return pl.pallas_call(
        _dec_kernel,
        in_specs=[
            pl.BlockSpec((8, 512), lambda: (0, 0)),
            pl.BlockSpec((8, 512), lambda: (0, 0)),
            pl.BlockSpec((1024, 1024), lambda: (0, 0)),
            pl.BlockSpec((1, 1024), lambda: (0, 0)),
            pl.BlockSpec((1024, 128), lambda: (0, 0)),
            pl.BlockSpec((1, 128), lambda: (0, 0)),
        ],
        out_specs=pl.BlockSpec((8, 128), lambda: (0, 0)),
        out_shape=jax.ShapeDtypeStruct((8, 128), jnp.float32),
    )(zmax, zsum, wd1, bd1.reshape(1, 1024), wd2, bd2.reshape(1, 128))


def _sc_deg(dst3, ew3):
    d = jnp.zeros((NTILE, NPAD), jnp.float32)
    return d.at[jnp.repeat(jnp.arange(NTILE), NB * KB).reshape(dst3.shape),
                dst3].add(ew3)


def _sc_spmm(xt, src3, dst3, ew3):
    nc = xt.shape[0]
    p = jnp.zeros((NPAD, nc * 128), jnp.float32)
    feat = jnp.transpose(xt, (1, 0, 2)).reshape(NPAD, nc * 128)
    p = p.at[dst3.reshape(-1)].add(ew3.reshape(-1, 1) * feat[src3.reshape(-1)])
    p = jnp.transpose(p.reshape(NPAD, nc, 128), (1, 0, 2))
    return jnp.stack([p, jnp.zeros_like(p)])


def kernel(x, edge_weight, W1, b1, W2, b2, W3, b3, Wmu, bmu, Wlv, blv,
           Wd1, bd1, Wd2, bd2, edge_index, beta):
    src = edge_index[0]
    dst = edge_index[1]
    pad = EPAD - E
    src3 = jnp.concatenate([src, jnp.zeros((pad,), src.dtype)]).reshape(NTILE, NB, KB)
    dst3 = jnp.concatenate([dst, jnp.zeros((pad,), dst.dtype)]).reshape(NTILE, NB, KB)
    ew3 = jnp.concatenate([edge_weight, jnp.zeros((pad,), edge_weight.dtype)]).reshape(NTILE, NB, KB)
    x_pad = jnp.pad(x, ((0, NPAD - N), (0, 0)))
    eps = jax.random.normal(jax.random.key(42), (N, 512), jnp.float32) * 0.01
    eps_pad = jnp.pad(eps, ((0, NPAD - N), (0, 0)))
    sb = jnp.asarray(0.5 * beta, jnp.float32).reshape(1, 1)

    degp = _sc_deg(dst3, ew3)
    dis, xt1 = _tc_prep(degp, x_pad)

    p1 = _sc_spmm(xt1.reshape(1, NPAD, 128), src3, dst3, ew3)
    h1 = _tc_post(p1, xt1.reshape(1, NPAD, 128), dis, b1, w=W1)
    xt2 = _tc_mm_chunk(h1, W2, dis)

    p2 = _sc_spmm(xt2, src3, dst3, ew3)
    h2 = _tc_post(p2, xt2, dis, b2)
    xt3 = _tc_mm_chunk(h2, W3, dis)

    p3 = _sc_spmm(xt3, src3, dst3, ew3)
    mu, lv, zmax, zsum = _tc_head(p3, xt3, dis, b3, Wmu, bmu, Wlv, blv,
                                  eps_pad, sb)
    recon = _tc_dec(zmax, zsum, Wd1, bd1, Wd2, bd2)
    return (recon[0:1], mu[:N], lv[:N])


# speedup vs baseline: 7.2388x; 3.4298x over previous
import functools

import jax
import jax.numpy as jnp
from jax import lax
from jax.experimental import pallas as pl
from jax.experimental.pallas import tpu as pltpu
from jax.experimental.pallas import tpu_sc as plsc

N = 10000
NPAD = 10240
E = 320000
NTILE = 32
KB = 128
NB = 79
EPAD = NTILE * NB * KB
MB = 1024
GRID_M = NPAD // MB


def _prep_kernel(degp_ref, x_ref, dis_ref, xt_ref):
    d = 1.0 + jnp.sum(degp_ref[...], axis=0)[:, None]
    dis = lax.rsqrt(d)
    dis_ref[...] = dis
    xt_ref[...] = x_ref[...] * dis


def _tc_prep(degp, x_pad):
    return pl.pallas_call(
        _prep_kernel,
        grid=(GRID_M,),
        in_specs=[
            pl.BlockSpec((NTILE, MB), lambda m: (0, m)),
            pl.BlockSpec((MB, 128), lambda m: (m, 0)),
        ],
        out_specs=[
            pl.BlockSpec((MB, 1), lambda m: (m, 0)),
            pl.BlockSpec((MB, 128), lambda m: (m, 0)),
        ],
        out_shape=[
            jax.ShapeDtypeStruct((NPAD, 1), jnp.float32),
            jax.ShapeDtypeStruct((NPAD, 128), jnp.float32),
        ],
    )(degp, x_pad)


def _post_kernel(nc, p_ref, xt_ref, dis_ref, w_ref, b_ref, h_ref):
    dis = dis_ref[...]
    cols = []
    for c in range(nc):
        cols.append(dis * (p_ref[0, c] + p_ref[1, c] + xt_ref[c]))
    y = jnp.concatenate(cols, axis=-1)
    if w_ref is not None:
        y = jnp.dot(y, w_ref[...], preferred_element_type=jnp.float32)
    y = y + b_ref[...]
    nrm = jnp.sqrt(jnp.sum(y * y, axis=-1, keepdims=True))
    y = y / jnp.clip(nrm, 1e-12, None)
    h_ref[...] = jnp.maximum(y, 0.0)


def _tc_post(p, xt, dis, b, w=None):
    nc = p.shape[1]
    fout = w.shape[1] if w is not None else nc * 128
    kern = functools.partial(_post_kernel, nc)
    in_specs = [
        pl.BlockSpec((2, nc, MB, 128), lambda m: (0, 0, m, 0)),
        pl.BlockSpec((nc, MB, 128), lambda m: (0, m, 0)),
        pl.BlockSpec((MB, 1), lambda m: (m, 0)),
    ]
    args = [p, xt, dis]
    if w is not None:
        in_specs.append(pl.BlockSpec(w.shape, lambda m: (0, 0)))
        args.append(w)
    else:
        kern = functools.partial(_post_kernel_nw, nc)
    in_specs.append(pl.BlockSpec((1, fout), lambda m: (0, 0)))
    args.append(b.reshape(1, fout))
    return pl.pallas_call(
        kern,
        grid=(GRID_M,),
        in_specs=in_specs,
        out_specs=pl.BlockSpec((MB, fout), lambda m: (m, 0)),
        out_shape=jax.ShapeDtypeStruct((NPAD, fout), jnp.float32),
    )(*args)


def _post_kernel_nw(nc, p_ref, xt_ref, dis_ref, b_ref, h_ref):
    _post_kernel(nc, p_ref, xt_ref, dis_ref, None, b_ref, h_ref)


def _mm_chunk_kernel(h_ref, w_ref, dis_ref, o_ref):
    g = jnp.dot(h_ref[...], w_ref[...], preferred_element_type=jnp.float32)
    o_ref[0] = dis_ref[...] * g


def _tc_mm_chunk(h, w, dis):
    kdim = h.shape[1]
    nc = w.shape[1] // 128
    return pl.pallas_call(
        _mm_chunk_kernel,
        grid=(GRID_M, nc),
        in_specs=[
            pl.BlockSpec((MB, kdim), lambda m, c: (m, 0)),
            pl.BlockSpec((kdim, 128), lambda m, c: (0, c)),
            pl.BlockSpec((MB, 1), lambda m, c: (m, 0)),
        ],
        out_specs=pl.BlockSpec((1, MB, 128), lambda m, c: (c, m, 0)),
        out_shape=jax.ShapeDtypeStruct((nc, NPAD, 128), jnp.float32),
    )(h, w, dis)


def _head_kernel(p_ref, xt_ref, dis_ref, b_ref, wmu_ref, bmu_ref, wlv_ref,
                 blv_ref, eps_ref, sb_ref, mu_ref, lv_ref, zmax_ref, zsum_ref):
    nc = 2
    dis = dis_ref[...]
    cols = []
    for c in range(nc):
        cols.append(dis * (p_ref[0, c] + p_ref[1, c] + xt_ref[c]))
    y = jnp.concatenate(cols, axis=-1) + b_ref[...]
    nrm = jnp.sqrt(jnp.sum(y * y, axis=-1, keepdims=True))
    h = jnp.maximum(y / jnp.clip(nrm, 1e-12, None), 0.0)
    mu = jnp.dot(h, wmu_ref[...], preferred_element_type=jnp.float32) + bmu_ref[...]
    lv = jnp.dot(h, wlv_ref[...], preferred_element_type=jnp.float32) + blv_ref[...]
    mu_ref[...] = mu
    lv_ref[...] = lv
    z = mu + eps_ref[...] * jnp.exp(sb_ref[0, 0] * lv)
    m = pl.program_id(0)
    rows = m * MB + lax.broadcasted_iota(jnp.int32, (MB, 1), 0)
    valid = rows < N
    zmax = jnp.max(jnp.where(valid, z, -jnp.inf), axis=0, keepdims=True)
    zsum = jnp.sum(jnp.where(valid, z, 0.0), axis=0, keepdims=True)
    zmax_b = jnp.broadcast_to(zmax, (8, zmax.shape[1]))
    zsum_b = jnp.broadcast_to(zsum, (8, zsum.shape[1]))

    @pl.when(m == 0)
    def _():
        zmax_ref[...] = zmax_b
        zsum_ref[...] = zsum_b

    @pl.when(m > 0)
    def _():
        zmax_ref[...] = jnp.maximum(zmax_ref[...], zmax_b)
        zsum_ref[...] = zsum_ref[...] + zsum_b


def _tc_head(p, xt, dis, b3, wmu, bmu, wlv, blv, eps_pad, sb):
    return pl.pallas_call(
        _head_kernel,
        grid=(GRID_M,),
        in_specs=[
            pl.BlockSpec((2, 2, MB, 128), lambda m: (0, 0, m, 0)),
            pl.BlockSpec((2, MB, 128), lambda m: (0, m, 0)),
            pl.BlockSpec((MB, 1), lambda m: (m, 0)),
            pl.BlockSpec((1, 256), lambda m: (0, 0)),
            pl.BlockSpec((256, 512), lambda m: (0, 0)),
            pl.BlockSpec((1, 512), lambda m: (0, 0)),
            pl.BlockSpec((256, 512), lambda m: (0, 0)),
            pl.BlockSpec((1, 512), lambda m: (0, 0)),
            pl.BlockSpec((MB, 512), lambda m: (m, 0)),
            pl.BlockSpec(memory_space=pltpu.SMEM),
        ],
        out_specs=[
            pl.BlockSpec((MB, 512), lambda m: (m, 0)),
            pl.BlockSpec((MB, 512), lambda m: (m, 0)),
            pl.BlockSpec((8, 512), lambda m: (0, 0)),
            pl.BlockSpec((8, 512), lambda m: (0, 0)),
        ],
        out_shape=[
            jax.ShapeDtypeStruct((NPAD, 512), jnp.float32),
            jax.ShapeDtypeStruct((NPAD, 512), jnp.float32),
            jax.ShapeDtypeStruct((8, 512), jnp.float32),
            jax.ShapeDtypeStruct((8, 512), jnp.float32),
        ],
    )(p, xt, dis, b3.reshape(1, 256), wmu, bmu.reshape(1, 512), wlv,
      blv.reshape(1, 512), eps_pad, sb)


def _dec_kernel(zmax_ref, zsum_ref, wd1_ref, bd1_ref, wd2_ref, bd2_ref, o_ref):
    rz = jnp.concatenate([zmax_ref[...], zsum_ref[...] / N], axis=-1)
    hd = jnp.dot(rz, wd1_ref[...], preferred_element_type=jnp.float32) + bd1_ref[...]
    hd = jnp.maximum(hd, 0.0)
    o = jnp.dot(hd, wd2_ref[...], preferred_element_type=jnp.float32) + bd2_ref[...]
    o_ref[...] = jax.nn.sigmoid(o)


def _tc_dec(zmax, zsum, wd1, bd1, wd2, bd2):
    return pl.pallas_call(
        _dec_kernel,
        in_specs=[
            pl.BlockSpec((8, 512), lambda: (0, 0)),
            pl.BlockSpec((8, 512), lambda: (0, 0)),
            pl.BlockSpec((1024, 1024), lambda: (0, 0)),
            pl.BlockSpec((1, 1024), lambda: (0, 0)),
            pl.BlockSpec((1024, 128), lambda: (0, 0)),
            pl.BlockSpec((1, 128), lambda: (0, 0)),
        ],
        out_specs=pl.BlockSpec((8, 128), lambda: (0, 0)),
        out_shape=jax.ShapeDtypeStruct((8, 128), jnp.float32),
    )(zmax, zsum, wd1, bd1.reshape(1, 1024), wd2, bd2.reshape(1, 128))


def _sc_mesh():
    return plsc.VectorSubcoreMesh(core_axis_name="c", subcore_axis_name="s")


def _sc_deg(dst3, ew3):
    @functools.partial(
        pl.kernel,
        out_type=jax.ShapeDtypeStruct((NTILE, NPAD), jnp.float32),
        mesh=_sc_mesh(),
        compiler_params=pltpu.CompilerParams(needs_layout_passes=False),
        scratch_types=[
            pltpu.VMEM((NB, KB), jnp.int32),
            pltpu.VMEM((NB, KB), jnp.float32),
            pltpu.VMEM((NPAD,), jnp.float32),
        ],
    )
    def k(dst_hbm, ew_hbm, out_hbm, idx_v, ew_v, deg_v):
        c = lax.axis_index("c")
        s = lax.axis_index("s")
        w = c * 16 + s
        pltpu.sync_copy(dst_hbm.at[w], idx_v)
        pltpu.sync_copy(ew_hbm.at[w], ew_v)

        def zbody(i, _):
            deg_v[pl.ds(i * 16, 16)] = jnp.zeros((16,), jnp.float32)
            return 0

        lax.fori_loop(0, NPAD // 16, zbody, 0, unroll=8)

        def body(b, _):
            for j in range(KB // 16):
                idx = idx_v[b, pl.ds(j * 16, 16)]
                vals = ew_v[b, pl.ds(j * 16, 16)]
                plsc.addupdate_scatter(deg_v, [idx], vals)
            return 0

        lax.fori_loop(0, NB, body, 0)
        pltpu.sync_copy(deg_v, out_hbm.at[w])

    return k(dst3, ew3)


def _sc_spmm(xt, src3, dst3, ew3):
    nc = xt.shape[0]

    @functools.partial(
        pl.kernel,
        out_type=jax.ShapeDtypeStruct((2, nc, NPAD, 128), jnp.float32),
        mesh=_sc_mesh(),
        compiler_params=pltpu.CompilerParams(needs_layout_passes=False),
        scratch_types=[
            pltpu.VMEM((NB, KB), jnp.int32),
            pltpu.VMEM((NB, KB), jnp.int32),
            pltpu.VMEM((NB, KB), jnp.float32),
            pltpu.VMEM((KB, 128), jnp.float32),
            pltpu.VMEM_SHARED((NPAD, 128), jnp.float32),
            pltpu.SemaphoreType.DMA,
        ],
    )
    def k(xt_hbm, src_hbm, dst_hbm, ew_hbm, out_hbm, src_v, dst_v, ew_v,
          rows_v, acc_sh, sem):
        c = lax.axis_index("c")
        s = lax.axis_index("s")
        w = c * 16 + s
        pltpu.sync_copy(src_hbm.at[w], src_v)
        pltpu.sync_copy(dst_hbm.at[w], dst_v)
        pltpu.sync_copy(ew_hbm.at[w], ew_v)
        rows_per_tile = NPAD // 16

        for chunk in range(nc):
            def zbody(i, _):
                for j in range(8):
                    rows_v[i, pl.ds(j * 16, 16)] = jnp.zeros((16,), jnp.float32)
                return 0

            lax.fori_loop(0, KB, zbody, 0)
            for r in range(rows_per_tile // KB):
                pltpu.sync_copy(rows_v,
                                acc_sh.at[pl.ds(s * rows_per_tile + r * KB, KB)])
            plsc.subcore_barrier()

            def body(b, _):
                pltpu.async_copy(xt_hbm.at[chunk].at[src_v.at[b]], rows_v,
                                 sem).wait()

                def mul(g, _):
                    wv = ew_v[b, pl.ds(g * 16, 16)]
                    for i in range(16):
                        kk = g * 16 + i
                        sc = wv[i]
                        for j in range(8):
                            rows_v[kk, pl.ds(j * 16, 16)] = (
                                rows_v[kk, pl.ds(j * 16, 16)] * sc)
                    return 0

                lax.fori_loop(0, KB // 16, mul, 0)
                pltpu.sync_copy(rows_v, acc_sh.at[dst_v.at[b]], add=True)
                return 0

            lax.fori_loop(0, NB, body, 0)
            plsc.subcore_barrier()
            for r in range(rows_per_tile // KB):
                off = s * rows_per_tile + r * KB
                pltpu.sync_copy(acc_sh.at[pl.ds(off, KB)], rows_v)
                pltpu.sync_copy(rows_v, out_hbm.at[c, chunk].at[pl.ds(off, KB)])
            plsc.subcore_barrier()

    return k(xt, src3, dst3, ew3)


def kernel(x, edge_weight, W1, b1, W2, b2, W3, b3, Wmu, bmu, Wlv, blv,
           Wd1, bd1, Wd2, bd2, edge_index, beta):
    src = edge_index[0]
    dst = edge_index[1]
    pad = EPAD - E
    src3 = jnp.concatenate([src, jnp.zeros((pad,), src.dtype)]).reshape(NTILE, NB, KB)
    dst3 = jnp.concatenate([dst, jnp.zeros((pad,), dst.dtype)]).reshape(NTILE, NB, KB)
    ew3 = jnp.concatenate([edge_weight, jnp.zeros((pad,), edge_weight.dtype)]).reshape(NTILE, NB, KB)
    x_pad = jnp.pad(x, ((0, NPAD - N), (0, 0)))
    eps = jax.random.normal(jax.random.key(42), (N, 512), jnp.float32) * 0.01
    eps_pad = jnp.pad(eps, ((0, NPAD - N), (0, 0)))
    sb = jnp.asarray(0.5 * beta, jnp.float32).reshape(1, 1)

    degp = _sc_deg(dst3, ew3)
    dis, xt1 = _tc_prep(degp, x_pad)

    p1 = _sc_spmm(xt1.reshape(1, NPAD, 128), src3, dst3, ew3)
    h1 = _tc_post(p1, xt1.reshape(1, NPAD, 128), dis, b1, w=W1)
    xt2 = _tc_mm_chunk(h1, W2, dis)

    p2 = _sc_spmm(xt2, src3, dst3, ew3)
    h2 = _tc_post(p2, xt2, dis, b2)
    xt3 = _tc_mm_chunk(h2, W3, dis)

    p3 = _sc_spmm(xt3, src3, dst3, ew3)
    mu, lv, zmax, zsum = _tc_head(p3, xt3, dis, b3, Wmu, bmu, Wlv, blv,
                                  eps_pad, sb)
    recon = _tc_dec(zmax, zsum, Wd1, bd1, Wd2, bd2)
    return (recon[0:1], mu[:N], lv[:N])
